# Initial kernel scaffold; baseline (speedup 1.0000x reference)
import jax
import jax.numpy as jnp
from jax.experimental import pallas as pl


def kernel(x, Wqkv, bqkv, Wo, bo, g1, be1, g2, be2, Wg, W1, b1, W2, b2):
    raise NotImplementedError("write your pallas kernel here")


# speedup vs baseline: 3.9571x; 3.9571x over previous
import functools

import jax
import jax.numpy as jnp
from jax import lax
from jax.experimental import pallas as pl
from jax.experimental.pallas import tpu as pltpu
import jax.experimental.pallas.tpu_sc as plsc

S, D, H, E, FF = 2048, 1024, 4, 8, 4096
DH = D // H
TM = 256
MAX_TILES = S // TM + E - 1
MPAD = MAX_TILES * TM
FFB = 512
NF = FF // FFB
NQ = 4
BQ = S // NQ

NC, NS = 2, 16
NW = NC * NS
TW = S // NW


def _qkv_body(x_ref, w_ref, b_ref, o_ref):
    o_ref[...] = lax.dot_general(
        x_ref[...], w_ref[...], (((1,), (1,)), ((), ())),
        preferred_element_type=jnp.float32) + b_ref[...]


def _qkv_proj(x2d, Wqkv, bqkv):
    return pl.pallas_call(
        _qkv_body,
        grid=(6,),
        in_specs=[
            pl.BlockSpec((S, D), lambda j: (0, 0)),
            pl.BlockSpec((FFB, D), lambda j: (j, 0)),
            pl.BlockSpec((1, FFB), lambda j: (0, j)),
        ],
        out_specs=pl.BlockSpec((S, FFB), lambda j: (0, j)),
        out_shape=jax.ShapeDtypeStruct((S, 3 * D), jnp.float32),
    )(x2d, Wqkv, bqkv.reshape(1, 3 * D))


def _attn_body(q_ref, k_ref, v_ref, o_ref):
    s = lax.dot_general(q_ref[...], k_ref[...], (((1,), (1,)), ((), ())),
                        preferred_element_type=jnp.float32)
    s = s * (1.0 / (DH ** 0.5))
    m = jnp.max(s, axis=-1, keepdims=True)
    p = jnp.exp(s - m)
    p = p / jnp.sum(p, axis=-1, keepdims=True)
    o_ref[...] = jnp.dot(p, v_ref[...], preferred_element_type=jnp.float32)


def _attention(qkv):
    return pl.pallas_call(
        _attn_body,
        grid=(H, NQ),
        in_specs=[
            pl.BlockSpec((BQ, DH), lambda h, q: (q, h)),
            pl.BlockSpec((S, DH), lambda h, q: (0, H + h)),
            pl.BlockSpec((S, DH), lambda h, q: (0, 2 * H + h)),
        ],
        out_specs=pl.BlockSpec((BQ, DH), lambda h, q: (q, h)),
        out_shape=jax.ShapeDtypeStruct((S, D), jnp.float32),
    )(qkv, qkv, qkv)


def _ln(s, g, b):
    mu = jnp.mean(s, axis=-1, keepdims=True)
    c = s - mu
    var = jnp.mean(c * c, axis=-1, keepdims=True)
    return c * lax.rsqrt(var + 1e-5) * g + b


def _post_attn_body(a_ref, wo_ref, bo_ref, x_ref, g1_ref, b1_ref, wg_ref,
                    t_ref, i_ref, oh_ref):
    proj = lax.dot_general(a_ref[...], wo_ref[...], (((1,), (1,)), ((), ())),
                           preferred_element_type=jnp.float32) + bo_ref[...]
    t = _ln(x_ref[...] + proj, g1_ref[...], b1_ref[...])
    t_ref[...] = t
    logits = jnp.dot(t, wg_ref[...], preferred_element_type=jnp.float32)
    m = logits[:, 0:1]
    mi = jnp.zeros_like(m, dtype=jnp.int32)
    for e in range(1, E):
        le = logits[:, e:e + 1]
        c = le > m
        m = jnp.where(c, le, m)
        mi = jnp.where(c, e, mi)
    i_ref[...] = mi.reshape(1, 1, TM)
    oh_ref[...] = (mi == jax.lax.broadcasted_iota(jnp.int32, (TM, E), 1)
                   ).astype(jnp.float32)


def _post_attn(attn, Wo, bo, x2d, g1, be1, Wg):
    return pl.pallas_call(
        _post_attn_body,
        grid=(S // TM,),
        in_specs=[
            pl.BlockSpec((TM, D), lambda i: (i, 0)),
            pl.BlockSpec((D, D), lambda i: (0, 0)),
            pl.BlockSpec((1, D), lambda i: (0, 0)),
            pl.BlockSpec((TM, D), lambda i: (i, 0)),
            pl.BlockSpec((1, D), lambda i: (0, 0)),
            pl.BlockSpec((1, D), lambda i: (0, 0)),
            pl.BlockSpec((D, E), lambda i: (0, 0)),
        ],
        out_specs=[
            pl.BlockSpec((TM, D), lambda i: (i, 0)),
            pl.BlockSpec((1, 1, TM), lambda i: (i, 0, 0)),
            pl.BlockSpec((TM, E), lambda i: (i, 0)),
        ],
        out_shape=[
            jax.ShapeDtypeStruct((S, D), jnp.float32),
            jax.ShapeDtypeStruct((S // TM, 1, TM), jnp.int32),
            jax.ShapeDtypeStruct((S, E), jnp.float32),
        ],
    )(attn, Wo, bo.reshape(1, D), x2d, g1.reshape(1, D), be1.reshape(1, D), Wg)


def _rank_body(oh_ref, d_ref, meta_ref):
    i = pl.program_id(0)
    oh = oh_ref[...]
    sizes = jnp.sum(oh, axis=0, keepdims=True)
    nt = jnp.ceil(sizes * (1.0 / TM))
    padded = nt * TM
    ps_parts = []
    run = jnp.zeros((1, 1), jnp.float32)
    rm_parts = []
    rmax = jnp.zeros((1, 1), jnp.float32)
    for e in range(E):
        ps_parts.append(run)
        run = run + padded[:, e:e + 1]
        ne_e = jnp.minimum(sizes[:, e:e + 1], 1.0)
        rmax = jnp.maximum(rmax, ne_e * e)
        rm_parts.append(rmax)
    ps = jnp.concatenate(ps_parts, axis=1)
    rm = jnp.concatenate(rm_parts, axis=1)
    ne = jnp.minimum(sizes, 1.0)

    rows = (i * TM + jax.lax.broadcasted_iota(jnp.int32, (TM, S), 0))
    cols = jax.lax.broadcasted_iota(jnp.int32, (TM, S), 1)
    tri = (cols < rows).astype(jnp.float32)
    rank = jnp.dot(tri, oh, preferred_element_type=jnp.float32)

    blk = oh_ref[pl.ds(i * TM, TM), :]
    dest = jnp.sum(blk * (ps + rank), axis=1)
    d_ref[...] = dest.astype(jnp.int32).reshape(1, 1, TM)

    @pl.when(i == 0)
    def _():
        meta_ref[...] = jnp.concatenate([rm, ps, nt, ne], axis=1
                                        ).astype(jnp.int32)


def _rank(oh):
    return pl.pallas_call(
        _rank_body,
        grid=(S // TM,),
        in_specs=[pl.BlockSpec((S, E), lambda i: (0, 0))],
        out_specs=[
            pl.BlockSpec((1, 1, TM), lambda i: (i, 0, 0)),
            pl.BlockSpec((1, 4 * E), lambda i: (0, 0)),
        ],
        out_shape=[
            jax.ShapeDtypeStruct((S // TM, 1, TM), jnp.int32),
            jax.ShapeDtypeStruct((1, 4 * E), jnp.int32),
        ],
    )(oh)


def _scatter_body(dest_hbm, t_hbm, xs_hbm, destv, rows, sem):
    wid = lax.axis_index("s") * NC + lax.axis_index("c")
    pltpu.sync_copy(dest_hbm.at[pl.ds(wid * TW, TW)], destv)
    pltpu.sync_copy(t_hbm.at[pl.ds(wid * TW, TW), :], rows)
    pltpu.async_copy(rows, xs_hbm.at[destv], sem).wait()


def _scatter(dest, t):
    mesh = plsc.VectorSubcoreMesh(core_axis_name="c", subcore_axis_name="s")
    f = pl.kernel(
        _scatter_body,
        out_type=jax.ShapeDtypeStruct((MPAD, D), jnp.float32),
        mesh=mesh,
        scratch_types=[
            pltpu.VMEM((TW,), jnp.int32),
            pltpu.VMEM((TW, D), jnp.float32),
            pltpu.SemaphoreType.DMA,
        ],
    )
    return f(dest, t)


_SQRT_HALF = 0.7071067811865476


def _gelu(h):
    return 0.5 * h * (1.0 + lax.erf(h * _SQRT_HALF))


def _ffn_body(rm_ref, ps_ref, nt_ref, ne_ref,
              x_ref, w1_ref, b1_ref, w2_ref, b2_ref, g2_ref, be2_ref, y_ref):
    e = pl.program_id(0)
    f = pl.program_id(1)
    nt_e = nt_ref[e]
    row0 = ps_ref[e]
    w1 = w1_ref[0]
    w2 = w2_ref[0]
    b1r = b1_ref[0]
    b2r = b2_ref[0]

    def tile(i, last):
        r = pl.multiple_of(row0 + i * TM, TM)
        xt = x_ref[pl.ds(r, TM), :]
        h = _gelu(jnp.dot(xt, w1, preferred_element_type=jnp.float32) + b1r)
        part = jnp.dot(h, w2, preferred_element_type=jnp.float32)
        acc = jnp.where(f == 0, b2r + part, y_ref[pl.ds(r, TM), :] + part)
        if last:
            y_ref[pl.ds(r, TM), :] = _ln(xt + acc, g2_ref[...], be2_ref[...])
        else:
            y_ref[pl.ds(r, TM), :] = acc

    @pl.when(f < NF - 1)
    def _():
        lax.fori_loop(0, nt_e, lambda i, c: (tile(i, False), c)[1], 0)

    @pl.when(f == NF - 1)
    def _():
        lax.fori_loop(0, nt_e, lambda i, c: (tile(i, True), c)[1], 0)


def _ffn(rm, ps, nt, ne, xs, W1, b1, W2, b2, g2, be2):
    def w_f(e, f, rm_ref, ps_ref, nt_ref, ne_ref):
        sp = jnp.where(e % 2 == 0, f, NF - 1 - f)
        sp0 = jnp.where(e % 2 == 0, 0, NF - 1)
        return jnp.where(ne_ref[e] == 1, sp, sp0)

    grid_spec = pltpu.PrefetchScalarGridSpec(
        num_scalar_prefetch=4,
        grid=(E, NF),
        in_specs=[
            pl.BlockSpec((MPAD, D), lambda e, f, rm, ps, nt, ne: (0, 0)),
            pl.BlockSpec((1, D, FFB),
                         lambda e, f, rm, ps, nt, ne:
                         (rm[e], 0, w_f(e, f, rm, ps, nt, ne))),
            pl.BlockSpec((1, 1, FFB),
                         lambda e, f, rm, ps, nt, ne:
                         (rm[e], 0, w_f(e, f, rm, ps, nt, ne))),
            pl.BlockSpec((1, FFB, D),
                         lambda e, f, rm, ps, nt, ne:
                         (rm[e], w_f(e, f, rm, ps, nt, ne), 0)),
            pl.BlockSpec((1, 1, D), lambda e, f, rm, ps, nt, ne: (rm[e], 0, 0)),
            pl.BlockSpec((1, D), lambda e, f, rm, ps, nt, ne: (0, 0)),
            pl.BlockSpec((1, D), lambda e, f, rm, ps, nt, ne: (0, 0)),
        ],
        out_specs=pl.BlockSpec((MPAD, D), lambda e, f, rm, ps, nt, ne: (0, 0)),
    )
    return pl.pallas_call(
        _ffn_body,
        grid_spec=grid_spec,
        out_shape=jax.ShapeDtypeStruct((MPAD, D), jnp.float32),
        compiler_params=pltpu.CompilerParams(
            vmem_limit_bytes=100 * 1024 * 1024),
    )(rm, ps, nt, ne, xs, W1, b1.reshape(E, 1, FF), W2, b2.reshape(E, 1, D),
      g2.reshape(1, D), be2.reshape(1, D))


def _unsort_body(dest_hbm, src_hbm, out_hbm, destv, rows, sem):
    wid = lax.axis_index("s") * NC + lax.axis_index("c")
    pltpu.sync_copy(dest_hbm.at[pl.ds(wid * TW, TW)], destv)
    pltpu.async_copy(src_hbm.at[destv], rows, sem).wait()
    pltpu.sync_copy(rows, out_hbm.at[pl.ds(wid * TW, TW), :])


def _unsort(dest, ys):
    mesh = plsc.VectorSubcoreMesh(core_axis_name="c", subcore_axis_name="s")
    f = pl.kernel(
        _unsort_body,
        out_type=jax.ShapeDtypeStruct((S, D), jnp.float32),
        mesh=mesh,
        scratch_types=[
            pltpu.VMEM((TW,), jnp.int32),
            pltpu.VMEM((TW, D), jnp.float32),
            pltpu.SemaphoreType.DMA,
        ],
    )
    return f(dest, ys)


def kernel(x, Wqkv, bqkv, Wo, bo, g1, be1, g2, be2, Wg, W1, b1, W2, b2):
    x2d = x.reshape(S, D)
    qkv = _qkv_proj(x2d, Wqkv, bqkv)
    attn = _attention(qkv)
    t, idx3, oh = _post_attn(attn, Wo, bo, x2d, g1, be1, Wg)

    dest3, meta = _rank(oh)
    dest = dest3.reshape(S)
    xs = _scatter(dest, t)
    rm, ps, nt, ne = (meta[0, 0:E], meta[0, E:2 * E], meta[0, 2 * E:3 * E],
                      meta[0, 3 * E:4 * E])

    ys = _ffn(rm, ps, nt, ne, xs, W1, b1, W2, b2, g2, be2)
    out2 = _unsort(dest, ys)
    return out2.reshape(S, 1, D)


# speedup vs baseline: 3.9828x; 1.0065x over previous
import functools

import jax
import jax.numpy as jnp
from jax import lax
from jax.experimental import pallas as pl
from jax.experimental.pallas import tpu as pltpu
import jax.experimental.pallas.tpu_sc as plsc

S, D, H, E, FF = 2048, 1024, 4, 8, 4096
DH = D // H
TM = 256
MAX_TILES = S // TM + E - 1
MPAD = MAX_TILES * TM
FFB = 512
NF = FF // FFB
NQ = 4
BQ = S // NQ

NC, NS = 2, 16
NW = NC * NS
TW = S // NW


def _qkv_body(x_ref, w_ref, b_ref, o_ref):
    o_ref[...] = lax.dot_general(
        x_ref[...], w_ref[...], (((1,), (1,)), ((), ())),
        preferred_element_type=jnp.float32) + b_ref[...]


def _qkv_proj(x2d, Wqkv, bqkv):
    return pl.pallas_call(
        _qkv_body,
        grid=(6,),
        in_specs=[
            pl.BlockSpec((S, D), lambda j: (0, 0)),
            pl.BlockSpec((FFB, D), lambda j: (j, 0)),
            pl.BlockSpec((1, FFB), lambda j: (0, j)),
        ],
        out_specs=pl.BlockSpec((S, FFB), lambda j: (0, j)),
        out_shape=jax.ShapeDtypeStruct((S, 3 * D), jnp.float32),
    )(x2d, Wqkv, bqkv.reshape(1, 3 * D))


def _attn_body(q_ref, k_ref, v_ref, o_ref):
    s = lax.dot_general(q_ref[...], k_ref[...], (((1,), (1,)), ((), ())),
                        preferred_element_type=jnp.float32)
    s = s * (1.0 / (DH ** 0.5))
    m = jnp.max(s, axis=-1, keepdims=True)
    p = jnp.exp(s - m)
    p = p / jnp.sum(p, axis=-1, keepdims=True)
    o_ref[...] = jnp.dot(p, v_ref[...], preferred_element_type=jnp.float32)


def _attention(qkv):
    return pl.pallas_call(
        _attn_body,
        grid=(H, NQ),
        in_specs=[
            pl.BlockSpec((BQ, DH), lambda h, q: (q, h)),
            pl.BlockSpec((S, DH), lambda h, q: (0, H + h)),
            pl.BlockSpec((S, DH), lambda h, q: (0, 2 * H + h)),
        ],
        out_specs=pl.BlockSpec((BQ, DH), lambda h, q: (q, h)),
        out_shape=jax.ShapeDtypeStruct((S, D), jnp.float32),
    )(qkv, qkv, qkv)


def _ln(s, g, b):
    mu = jnp.mean(s, axis=-1, keepdims=True)
    c = s - mu
    var = jnp.mean(c * c, axis=-1, keepdims=True)
    return c * lax.rsqrt(var + 1e-5) * g + b


def _post_attn_body(a_ref, wo_ref, bo_ref, x_ref, g1_ref, b1_ref, wg_ref,
                    t_ref, i_ref, oh_ref):
    proj = lax.dot_general(a_ref[...], wo_ref[...], (((1,), (1,)), ((), ())),
                           preferred_element_type=jnp.float32) + bo_ref[...]
    t = _ln(x_ref[...] + proj, g1_ref[...], b1_ref[...])
    t_ref[...] = t
    logits = jnp.dot(t, wg_ref[...], preferred_element_type=jnp.float32)
    m = logits[:, 0:1]
    mi = jnp.zeros_like(m, dtype=jnp.int32)
    for e in range(1, E):
        le = logits[:, e:e + 1]
        c = le > m
        m = jnp.where(c, le, m)
        mi = jnp.where(c, e, mi)
    i_ref[...] = mi.reshape(1, 1, TM)
    oh_ref[...] = (mi == jax.lax.broadcasted_iota(jnp.int32, (TM, E), 1)
                   ).astype(jnp.float32)


def _post_attn(attn, Wo, bo, x2d, g1, be1, Wg):
    return pl.pallas_call(
        _post_attn_body,
        grid=(S // TM,),
        in_specs=[
            pl.BlockSpec((TM, D), lambda i: (i, 0)),
            pl.BlockSpec((D, D), lambda i: (0, 0)),
            pl.BlockSpec((1, D), lambda i: (0, 0)),
            pl.BlockSpec((TM, D), lambda i: (i, 0)),
            pl.BlockSpec((1, D), lambda i: (0, 0)),
            pl.BlockSpec((1, D), lambda i: (0, 0)),
            pl.BlockSpec((D, E), lambda i: (0, 0)),
        ],
        out_specs=[
            pl.BlockSpec((TM, D), lambda i: (i, 0)),
            pl.BlockSpec((1, 1, TM), lambda i: (i, 0, 0)),
            pl.BlockSpec((TM, E), lambda i: (i, 0)),
        ],
        out_shape=[
            jax.ShapeDtypeStruct((S, D), jnp.float32),
            jax.ShapeDtypeStruct((S // TM, 1, TM), jnp.int32),
            jax.ShapeDtypeStruct((S, E), jnp.float32),
        ],
    )(attn, Wo, bo.reshape(1, D), x2d, g1.reshape(1, D), be1.reshape(1, D), Wg)


def _rank_body(oh_ref, d_ref, meta_ref):
    i = pl.program_id(0)
    oh = oh_ref[...]
    sizes = jnp.sum(oh, axis=0, keepdims=True)
    nt = jnp.ceil(sizes * (1.0 / TM))
    padded = nt * TM
    ps_parts = []
    run = jnp.zeros((1, 1), jnp.float32)
    rm_parts = []
    rmax = jnp.zeros((1, 1), jnp.float32)
    for e in range(E):
        ps_parts.append(run)
        run = run + padded[:, e:e + 1]
        ne_e = jnp.minimum(sizes[:, e:e + 1], 1.0)
        rmax = jnp.maximum(rmax, ne_e * e)
        rm_parts.append(rmax)
    ps = jnp.concatenate(ps_parts, axis=1)
    rm = jnp.concatenate(rm_parts, axis=1)
    ne = jnp.minimum(sizes, 1.0)

    rows = (i * TM + jax.lax.broadcasted_iota(jnp.int32, (TM, S), 0))
    cols = jax.lax.broadcasted_iota(jnp.int32, (TM, S), 1)
    tri = (cols < rows).astype(jnp.float32)
    rank = jnp.dot(tri, oh, preferred_element_type=jnp.float32)

    blk = oh_ref[pl.ds(i * TM, TM), :]
    dest = jnp.sum(blk * (ps + rank), axis=1)
    d_ref[...] = dest.astype(jnp.int32).reshape(1, 1, TM)

    @pl.when(i == 0)
    def _():
        meta_ref[...] = jnp.concatenate([rm, ps, nt, ne], axis=1
                                        ).astype(jnp.int32)


def _rank(oh):
    return pl.pallas_call(
        _rank_body,
        grid=(S // TM,),
        in_specs=[pl.BlockSpec((S, E), lambda i: (0, 0))],
        out_specs=[
            pl.BlockSpec((1, 1, TM), lambda i: (i, 0, 0)),
            pl.BlockSpec((1, 4 * E), lambda i: (0, 0)),
        ],
        out_shape=[
            jax.ShapeDtypeStruct((S // TM, 1, TM), jnp.int32),
            jax.ShapeDtypeStruct((1, 4 * E), jnp.int32),
        ],
    )(oh)


def _scatter_body(dest_hbm, t_hbm, xs_hbm, destv, rows, sem):
    wid = lax.axis_index("s") * NC + lax.axis_index("c")
    pltpu.sync_copy(dest_hbm.at[pl.ds(wid * TW, TW)], destv)
    pltpu.sync_copy(t_hbm.at[pl.ds(wid * TW, TW), :], rows)
    pltpu.async_copy(rows, xs_hbm.at[destv], sem).wait()


def _scatter(dest, t):
    mesh = plsc.VectorSubcoreMesh(core_axis_name="c", subcore_axis_name="s")
    f = pl.kernel(
        _scatter_body,
        out_type=jax.ShapeDtypeStruct((MPAD, D), jnp.float32),
        mesh=mesh,
        scratch_types=[
            pltpu.VMEM((TW,), jnp.int32),
            pltpu.VMEM((TW, D), jnp.float32),
            pltpu.SemaphoreType.DMA,
        ],
    )
    return f(dest, t)


_SQRT_HALF = 0.7071067811865476


def _gelu(h):
    return 0.5 * h * (1.0 + lax.erf(h * _SQRT_HALF))


def _ffn_body(rm_ref, ps_ref, nt_ref, ne_ref,
              x_ref, w1_ref, b1_ref, w2_ref, b2_ref, g2_ref, be2_ref, y_ref):
    e = pl.program_id(0)
    f = pl.program_id(1)
    nt_e = nt_ref[e]
    row0 = ps_ref[e]
    w1 = w1_ref[0]
    w2 = w2_ref[0]
    b1r = b1_ref[0]
    b2r = b2_ref[0]

    w1b = w1.astype(jnp.bfloat16)
    w2b = w2.astype(jnp.bfloat16)

    def tile(i, last):
        r = pl.multiple_of(row0 + i * TM, TM)
        xt = x_ref[pl.ds(r, TM), :]
        h = _gelu(jnp.dot(xt.astype(jnp.bfloat16), w1b,
                          preferred_element_type=jnp.float32) + b1r)
        part = jnp.dot(h.astype(jnp.bfloat16), w2b,
                       preferred_element_type=jnp.float32)
        acc = jnp.where(f == 0, b2r + part, y_ref[pl.ds(r, TM), :] + part)
        if last:
            y_ref[pl.ds(r, TM), :] = _ln(xt + acc, g2_ref[...], be2_ref[...])
        else:
            y_ref[pl.ds(r, TM), :] = acc

    @pl.when(f < NF - 1)
    def _():
        lax.fori_loop(0, nt_e, lambda i, c: (tile(i, False), c)[1], 0)

    @pl.when(f == NF - 1)
    def _():
        lax.fori_loop(0, nt_e, lambda i, c: (tile(i, True), c)[1], 0)


def _ffn(rm, ps, nt, ne, xs, W1, b1, W2, b2, g2, be2):
    def w_f(e, f, rm_ref, ps_ref, nt_ref, ne_ref):
        sp = jnp.where(e % 2 == 0, f, NF - 1 - f)
        sp0 = jnp.where(e % 2 == 0, 0, NF - 1)
        return jnp.where(ne_ref[e] == 1, sp, sp0)

    grid_spec = pltpu.PrefetchScalarGridSpec(
        num_scalar_prefetch=4,
        grid=(E, NF),
        in_specs=[
            pl.BlockSpec((MPAD, D), lambda e, f, rm, ps, nt, ne: (0, 0)),
            pl.BlockSpec((1, D, FFB),
                         lambda e, f, rm, ps, nt, ne:
                         (rm[e], 0, w_f(e, f, rm, ps, nt, ne))),
            pl.BlockSpec((1, 1, FFB),
                         lambda e, f, rm, ps, nt, ne:
                         (rm[e], 0, w_f(e, f, rm, ps, nt, ne))),
            pl.BlockSpec((1, FFB, D),
                         lambda e, f, rm, ps, nt, ne:
                         (rm[e], w_f(e, f, rm, ps, nt, ne), 0)),
            pl.BlockSpec((1, 1, D), lambda e, f, rm, ps, nt, ne: (rm[e], 0, 0)),
            pl.BlockSpec((1, D), lambda e, f, rm, ps, nt, ne: (0, 0)),
            pl.BlockSpec((1, D), lambda e, f, rm, ps, nt, ne: (0, 0)),
        ],
        out_specs=pl.BlockSpec((MPAD, D), lambda e, f, rm, ps, nt, ne: (0, 0)),
    )
    return pl.pallas_call(
        _ffn_body,
        grid_spec=grid_spec,
        out_shape=jax.ShapeDtypeStruct((MPAD, D), jnp.float32),
        compiler_params=pltpu.CompilerParams(
            vmem_limit_bytes=100 * 1024 * 1024),
    )(rm, ps, nt, ne, xs, W1, b1.reshape(E, 1, FF), W2, b2.reshape(E, 1, D),
      g2.reshape(1, D), be2.reshape(1, D))


def _unsort_body(dest_hbm, src_hbm, out_hbm, destv, rows, sem):
    wid = lax.axis_index("s") * NC + lax.axis_index("c")
    pltpu.sync_copy(dest_hbm.at[pl.ds(wid * TW, TW)], destv)
    pltpu.async_copy(src_hbm.at[destv], rows, sem).wait()
    pltpu.sync_copy(rows, out_hbm.at[pl.ds(wid * TW, TW), :])


def _unsort(dest, ys):
    mesh = plsc.VectorSubcoreMesh(core_axis_name="c", subcore_axis_name="s")
    f = pl.kernel(
        _unsort_body,
        out_type=jax.ShapeDtypeStruct((S, D), jnp.float32),
        mesh=mesh,
        scratch_types=[
            pltpu.VMEM((TW,), jnp.int32),
            pltpu.VMEM((TW, D), jnp.float32),
            pltpu.SemaphoreType.DMA,
        ],
    )
    return f(dest, ys)


def kernel(x, Wqkv, bqkv, Wo, bo, g1, be1, g2, be2, Wg, W1, b1, W2, b2):
    x2d = x.reshape(S, D)
    qkv = _qkv_proj(x2d, Wqkv, bqkv)
    attn = _attention(qkv)
    t, idx3, oh = _post_attn(attn, Wo, bo, x2d, g1, be1, Wg)

    dest3, meta = _rank(oh)
    dest = dest3.reshape(S)
    xs = _scatter(dest, t)
    rm, ps, nt, ne = (meta[0, 0:E], meta[0, E:2 * E], meta[0, 2 * E:3 * E],
                      meta[0, 3 * E:4 * E])

    ys = _ffn(rm, ps, nt, ne, xs, W1, b1, W2, b2, g2, be2)
    out2 = _unsort(dest, ys)
    return out2.reshape(S, 1, D)


# speedup vs baseline: 4.5640x; 1.1459x over previous
import functools

import jax
import jax.numpy as jnp
from jax import lax
from jax.experimental import pallas as pl
from jax.experimental.pallas import tpu as pltpu
import jax.experimental.pallas.tpu_sc as plsc

S, D, H, E, FF = 2048, 1024, 4, 8, 4096
DH = D // H
TM = 256
MAX_TILES = S // TM + E - 1
MPAD = MAX_TILES * TM
FFB = 1024
NF = FF // FFB
NQ = 4
BQ = S // NQ

NC, NS = 2, 16
NW = NC * NS
TW = S // NW


def _qkv_body(x_ref, w_ref, b_ref, o_ref):
    o_ref[...] = lax.dot_general(
        x_ref[...], w_ref[...], (((1,), (1,)), ((), ())),
        preferred_element_type=jnp.float32) + b_ref[...]


def _qkv_proj(x2d, Wqkv, bqkv):
    return pl.pallas_call(
        _qkv_body,
        grid=(6,),
        in_specs=[
            pl.BlockSpec((S, D), lambda j: (0, 0)),
            pl.BlockSpec((FFB, D), lambda j: (j, 0)),
            pl.BlockSpec((1, FFB), lambda j: (0, j)),
        ],
        out_specs=pl.BlockSpec((S, FFB), lambda j: (0, j)),
        out_shape=jax.ShapeDtypeStruct((S, 3 * D), jnp.float32),
    )(x2d, Wqkv, bqkv.reshape(1, 3 * D))


def _attn_body(q_ref, k_ref, v_ref, o_ref):
    s = lax.dot_general(q_ref[...], k_ref[...], (((1,), (1,)), ((), ())),
                        preferred_element_type=jnp.float32)
    s = s * (1.0 / (DH ** 0.5))
    m = jnp.max(s, axis=-1, keepdims=True)
    p = jnp.exp(s - m)
    p = p / jnp.sum(p, axis=-1, keepdims=True)
    o_ref[...] = jnp.dot(p, v_ref[...], preferred_element_type=jnp.float32)


def _attention(qkv):
    return pl.pallas_call(
        _attn_body,
        grid=(H, NQ),
        in_specs=[
            pl.BlockSpec((BQ, DH), lambda h, q: (q, h)),
            pl.BlockSpec((S, DH), lambda h, q: (0, H + h)),
            pl.BlockSpec((S, DH), lambda h, q: (0, 2 * H + h)),
        ],
        out_specs=pl.BlockSpec((BQ, DH), lambda h, q: (q, h)),
        out_shape=jax.ShapeDtypeStruct((S, D), jnp.float32),
    )(qkv, qkv, qkv)


def _ln(s, g, b):
    mu = jnp.mean(s, axis=-1, keepdims=True)
    c = s - mu
    var = jnp.mean(c * c, axis=-1, keepdims=True)
    return c * lax.rsqrt(var + 1e-5) * g + b


def _post_attn_body(a_ref, wo_ref, bo_ref, x_ref, g1_ref, b1_ref, wg_ref,
                    t_ref, i_ref, oh_ref):
    proj = lax.dot_general(a_ref[...], wo_ref[...], (((1,), (1,)), ((), ())),
                           preferred_element_type=jnp.float32) + bo_ref[...]
    t = _ln(x_ref[...] + proj, g1_ref[...], b1_ref[...])
    t_ref[...] = t
    logits = jnp.dot(t, wg_ref[...], preferred_element_type=jnp.float32)
    m = logits[:, 0:1]
    mi = jnp.zeros_like(m, dtype=jnp.int32)
    for e in range(1, E):
        le = logits[:, e:e + 1]
        c = le > m
        m = jnp.where(c, le, m)
        mi = jnp.where(c, e, mi)
    i_ref[...] = mi.reshape(1, 1, TM)
    oh_ref[...] = (mi == jax.lax.broadcasted_iota(jnp.int32, (TM, E), 1)
                   ).astype(jnp.float32)


def _post_attn(attn, Wo, bo, x2d, g1, be1, Wg):
    return pl.pallas_call(
        _post_attn_body,
        grid=(S // TM,),
        in_specs=[
            pl.BlockSpec((TM, D), lambda i: (i, 0)),
            pl.BlockSpec((D, D), lambda i: (0, 0)),
            pl.BlockSpec((1, D), lambda i: (0, 0)),
            pl.BlockSpec((TM, D), lambda i: (i, 0)),
            pl.BlockSpec((1, D), lambda i: (0, 0)),
            pl.BlockSpec((1, D), lambda i: (0, 0)),
            pl.BlockSpec((D, E), lambda i: (0, 0)),
        ],
        out_specs=[
            pl.BlockSpec((TM, D), lambda i: (i, 0)),
            pl.BlockSpec((1, 1, TM), lambda i: (i, 0, 0)),
            pl.BlockSpec((TM, E), lambda i: (i, 0)),
        ],
        out_shape=[
            jax.ShapeDtypeStruct((S, D), jnp.float32),
            jax.ShapeDtypeStruct((S // TM, 1, TM), jnp.int32),
            jax.ShapeDtypeStruct((S, E), jnp.float32),
        ],
    )(attn, Wo, bo.reshape(1, D), x2d, g1.reshape(1, D), be1.reshape(1, D), Wg)


def _rank_body(oh_ref, d_ref, meta_ref):
    i = pl.program_id(0)
    oh = oh_ref[...]
    sizes = jnp.sum(oh, axis=0, keepdims=True)
    nt = jnp.ceil(sizes * (1.0 / TM))
    padded = nt * TM
    ps_parts = []
    run = jnp.zeros((1, 1), jnp.float32)
    rm_parts = []
    rmax = jnp.zeros((1, 1), jnp.float32)
    for e in range(E):
        ps_parts.append(run)
        run = run + padded[:, e:e + 1]
        ne_e = jnp.minimum(sizes[:, e:e + 1], 1.0)
        rmax = jnp.maximum(rmax, ne_e * e)
        rm_parts.append(rmax)
    ps = jnp.concatenate(ps_parts, axis=1)
    rm = jnp.concatenate(rm_parts, axis=1)
    ne = jnp.minimum(sizes, 1.0)

    rows = (i * TM + jax.lax.broadcasted_iota(jnp.int32, (TM, S), 0))
    cols = jax.lax.broadcasted_iota(jnp.int32, (TM, S), 1)
    tri = (cols < rows).astype(jnp.float32)
    rank = jnp.dot(tri, oh, preferred_element_type=jnp.float32)

    blk = oh_ref[pl.ds(i * TM, TM), :]
    dest = jnp.sum(blk * (ps + rank), axis=1)
    d_ref[...] = dest.astype(jnp.int32).reshape(1, 1, TM)

    @pl.when(i == 0)
    def _():
        meta_ref[...] = jnp.concatenate([rm, ps, nt, ne], axis=1
                                        ).astype(jnp.int32)


def _rank(oh):
    return pl.pallas_call(
        _rank_body,
        grid=(S // TM,),
        in_specs=[pl.BlockSpec((S, E), lambda i: (0, 0))],
        out_specs=[
            pl.BlockSpec((1, 1, TM), lambda i: (i, 0, 0)),
            pl.BlockSpec((1, 4 * E), lambda i: (0, 0)),
        ],
        out_shape=[
            jax.ShapeDtypeStruct((S // TM, 1, TM), jnp.int32),
            jax.ShapeDtypeStruct((1, 4 * E), jnp.int32),
        ],
    )(oh)


def _scatter_body(dest_hbm, t_hbm, xs_hbm, destv, rows, sem):
    wid = lax.axis_index("s") * NC + lax.axis_index("c")
    pltpu.sync_copy(dest_hbm.at[pl.ds(wid * TW, TW)], destv)
    pltpu.sync_copy(t_hbm.at[pl.ds(wid * TW, TW), :], rows)
    pltpu.async_copy(rows, xs_hbm.at[destv], sem).wait()


def _scatter(dest, t):
    mesh = plsc.VectorSubcoreMesh(core_axis_name="c", subcore_axis_name="s")
    f = pl.kernel(
        _scatter_body,
        out_type=jax.ShapeDtypeStruct((MPAD, D), jnp.float32),
        mesh=mesh,
        scratch_types=[
            pltpu.VMEM((TW,), jnp.int32),
            pltpu.VMEM((TW, D), jnp.float32),
            pltpu.SemaphoreType.DMA,
        ],
    )
    return f(dest, t)


_SQRT_HALF = 0.7071067811865476


def _gelu(h):
    return 0.5 * h * (1.0 + lax.erf(h * _SQRT_HALF))


def _ffn_body(rm_ref, ps_ref, nt_ref, ne_ref,
              x_ref, w1_ref, b1_ref, w2_ref, b2_ref, g2_ref, be2_ref, y_ref):
    e = pl.program_id(0)
    f = pl.program_id(1)
    nt_e = nt_ref[e]
    row0 = ps_ref[e]
    w1 = w1_ref[0]
    w2 = w2_ref[0]
    b1r = b1_ref[0]
    b2r = b2_ref[0]

    w1b = w1.astype(jnp.bfloat16)
    w2b = w2.astype(jnp.bfloat16)

    def tile(i, last):
        r = pl.multiple_of(row0 + i * TM, TM)
        xt = x_ref[pl.ds(r, TM), :]
        h = _gelu(jnp.dot(xt.astype(jnp.bfloat16), w1b,
                          preferred_element_type=jnp.float32) + b1r)
        part = jnp.dot(h.astype(jnp.bfloat16), w2b,
                       preferred_element_type=jnp.float32)
        acc = jnp.where(f == 0, b2r + part, y_ref[pl.ds(r, TM), :] + part)
        if last:
            y_ref[pl.ds(r, TM), :] = _ln(xt + acc, g2_ref[...], be2_ref[...])
        else:
            y_ref[pl.ds(r, TM), :] = acc

    @pl.when(f < NF - 1)
    def _():
        lax.fori_loop(0, nt_e, lambda i, c: (tile(i, False), c)[1], 0)

    @pl.when(f == NF - 1)
    def _():
        lax.fori_loop(0, nt_e, lambda i, c: (tile(i, True), c)[1], 0)


def _ffn(rm, ps, nt, ne, xs, W1, b1, W2, b2, g2, be2):
    def w_f(e, f, rm_ref, ps_ref, nt_ref, ne_ref):
        sp = jnp.where(e % 2 == 0, f, NF - 1 - f)
        sp0 = jnp.where(e % 2 == 0, 0, NF - 1)
        return jnp.where(ne_ref[e] == 1, sp, sp0)

    grid_spec = pltpu.PrefetchScalarGridSpec(
        num_scalar_prefetch=4,
        grid=(E, NF),
        in_specs=[
            pl.BlockSpec((MPAD, D), lambda e, f, rm, ps, nt, ne: (0, 0)),
            pl.BlockSpec((1, D, FFB),
                         lambda e, f, rm, ps, nt, ne:
                         (rm[e], 0, w_f(e, f, rm, ps, nt, ne))),
            pl.BlockSpec((1, 1, FFB),
                         lambda e, f, rm, ps, nt, ne:
                         (rm[e], 0, w_f(e, f, rm, ps, nt, ne))),
            pl.BlockSpec((1, FFB, D),
                         lambda e, f, rm, ps, nt, ne:
                         (rm[e], w_f(e, f, rm, ps, nt, ne), 0)),
            pl.BlockSpec((1, 1, D), lambda e, f, rm, ps, nt, ne: (rm[e], 0, 0)),
            pl.BlockSpec((1, D), lambda e, f, rm, ps, nt, ne: (0, 0)),
            pl.BlockSpec((1, D), lambda e, f, rm, ps, nt, ne: (0, 0)),
        ],
        out_specs=pl.BlockSpec((MPAD, D), lambda e, f, rm, ps, nt, ne: (0, 0)),
    )
    return pl.pallas_call(
        _ffn_body,
        grid_spec=grid_spec,
        out_shape=jax.ShapeDtypeStruct((MPAD, D), jnp.float32),
        compiler_params=pltpu.CompilerParams(
            vmem_limit_bytes=100 * 1024 * 1024),
    )(rm, ps, nt, ne, xs, W1, b1.reshape(E, 1, FF), W2, b2.reshape(E, 1, D),
      g2.reshape(1, D), be2.reshape(1, D))


def _unsort_body(dest_hbm, src_hbm, out_hbm, destv, rows, sem):
    wid = lax.axis_index("s") * NC + lax.axis_index("c")
    pltpu.sync_copy(dest_hbm.at[pl.ds(wid * TW, TW)], destv)
    pltpu.async_copy(src_hbm.at[destv], rows, sem).wait()
    pltpu.sync_copy(rows, out_hbm.at[pl.ds(wid * TW, TW), :])


def _unsort(dest, ys):
    mesh = plsc.VectorSubcoreMesh(core_axis_name="c", subcore_axis_name="s")
    f = pl.kernel(
        _unsort_body,
        out_type=jax.ShapeDtypeStruct((S, D), jnp.float32),
        mesh=mesh,
        scratch_types=[
            pltpu.VMEM((TW,), jnp.int32),
            pltpu.VMEM((TW, D), jnp.float32),
            pltpu.SemaphoreType.DMA,
        ],
    )
    return f(dest, ys)


def kernel(x, Wqkv, bqkv, Wo, bo, g1, be1, g2, be2, Wg, W1, b1, W2, b2):
    x2d = x.reshape(S, D)
    qkv = _qkv_proj(x2d, Wqkv, bqkv)
    attn = _attention(qkv)
    t, idx3, oh = _post_attn(attn, Wo, bo, x2d, g1, be1, Wg)

    dest3, meta = _rank(oh)
    dest = dest3.reshape(S)
    xs = _scatter(dest, t)
    rm, ps, nt, ne = (meta[0, 0:E], meta[0, E:2 * E], meta[0, 2 * E:3 * E],
                      meta[0, 3 * E:4 * E])

    ys = _ffn(rm, ps, nt, ne, xs, W1, b1, W2, b2, g2, be2)
    out2 = _unsort(dest, ys)
    return out2.reshape(S, 1, D)
